# baseline (speedup 1.0000x reference)
import functools

import jax
import jax.numpy as jnp
from jax import lax
from jax.experimental import pallas as pl
from jax.experimental.pallas import tpu as pltpu
from jax.experimental.pallas import tpu_sc as plsc

_NC = 2
_NS = 16
_NW = _NC * _NS
_L = 16
_SUB = 4
_CH = _SUB * _L
_MARGIN = 1.0


def _plan(t):
    assert t % _L == 0
    ng_total = t // _L
    g_base = ng_total // _NW
    g_rem = ng_total % _NW
    maxg = g_base + (1 if g_rem else 0)
    rpw = -(-maxg // _SUB)
    if rpw % 2 == 0:
        rpw += 1
    iw = rpw * _CH
    return g_base, g_rem, rpw, iw


def _make_kernel(n_rows, d, t, npad):
    assert d % 32 == 0
    dp = d // 2
    g_base, g_rem, rpw, iw = _plan(t)
    mesh = plsc.VectorSubcoreMesh(core_axis_name="c", subcore_axis_name="s")

    @functools.partial(
        pl.kernel,
        mesh=mesh,
        compiler_params=pltpu.CompilerParams(use_tc_tiling_on_sc=False,
                                             needs_layout_passes=False),
        out_type=jax.ShapeDtypeStruct((_NW, _L), jnp.float32),
        scratch_types=[
            pltpu.VMEM((iw,), jnp.int32),
            pltpu.VMEM((iw,), jnp.int32),
            pltpu.VMEM((iw,), jnp.int32),
            pltpu.VMEM((2, _CH, dp), jnp.int32),
            pltpu.VMEM((2, _CH, dp), jnp.int32),
            pltpu.VMEM((2, _CH, dp), jnp.int32),
            pltpu.VMEM((npad,), jnp.float32),
            pltpu.VMEM((npad,), jnp.float32),
            pltpu.VMEM((_L,), jnp.float32),
            pltpu.SemaphoreType.DMA,
            pltpu.SemaphoreType.DMA,
        ],
    )
    def tl_kernel(h1, h2, h3, ia, ip, inn, norms, out,
                  ixa, ixp, ixn, ra, rp, rn, n2v, n3v, stage, sem0, sem1):
        wid = lax.axis_index("s") * _NC + lax.axis_index("c")
        ngroups = g_base + jnp.where(wid < g_rem, 1, 0)
        start = (g_base * wid + jnp.minimum(wid, g_rem)) * _L

        pltpu.sync_copy(ia.at[pl.ds(start, iw)], ixa)
        pltpu.sync_copy(ip.at[pl.ds(start, iw)], ixp)
        pltpu.sync_copy(inn.at[pl.ds(start, iw)], ixn)
        pltpu.sync_copy(norms.at[0], n2v)
        pltpu.sync_copy(norms.at[1], n3v)

        lanes = lax.iota(jnp.int32, _L)
        sems = (sem0, sem1)

        def fire(r, b):
            rb = r * _CH
            pltpu.async_copy(h1.at[ixa.at[pl.ds(rb, _CH)]], ra.at[b], sems[b])
            pltpu.async_copy(h2.at[ixp.at[pl.ds(rb, _CH)]], rp.at[b], sems[b])
            pltpu.async_copy(h3.at[ixn.at[pl.ds(rb, _CH)]], rn.at[b], sems[b])

        def drain(b):
            for buf in (ra, rp, rn):
                pltpu.make_async_copy(
                    h1.at[ixa.at[pl.ds(0, _CH)]], buf.at[b], sems[b]).wait()

        hi_mask = jnp.full((_L,), -65536, jnp.int32)

        def compute(r, b, total_v):
            for sub in range(_SUB):
                rows = lanes + (sub * _L)
                ipv = ixp[pl.ds(r * _CH + sub * _L, _L)]
                inv = ixn[pl.ds(r * _CH + sub * _L, _L)]
                g2 = plsc.load_gather(n2v, [ipv])
                g3 = plsc.load_gather(n3v, [inv])
                accs = [jnp.zeros((_L,), jnp.float32) for _ in range(4)]

                def d_body(db, accs, _rows=rows):
                    accs = list(accs)
                    basev = jnp.full((_L,), db * _L, jnp.int32)
                    def _halves(v):
                        hi = plsc.bitcast(v & hi_mask, jnp.float32)
                        lo = plsc.bitcast(v << 16, jnp.float32)
                        return hi, lo

                    for s in range(_L):
                        dv = basev + ((lanes + s) & (_L - 1))
                        a0, a1 = _halves(plsc.load_gather(ra.at[b], [_rows, dv]))
                        p0, p1 = _halves(plsc.load_gather(rp.at[b], [_rows, dv]))
                        n0, n1 = _halves(plsc.load_gather(rn.at[b], [_rows, dv]))
                        k = (s % 2) * 2
                        accs[k] = accs[k] + a0 * (p0 - n0)
                        accs[k + 1] = accs[k + 1] + a1 * (p1 - n1)
                    return tuple(accs)

                accs = lax.fori_loop(0, dp // _L, d_body, tuple(accs))
                dot = (accs[0] + accs[1]) + (accs[2] + accs[3])
                lossv = jnp.maximum(g2 - g3 - dot - dot + _MARGIN, 0.0)
                gate = (r * _SUB + sub < ngroups).astype(jnp.float32)
                total_v = total_v + lossv * gate
            return total_v

        fire(0, 0)

        def pair_body(k, total_v):
            r0 = 2 * k
            fire(r0 + 1, 1)
            drain(0)
            total_v = compute(r0, 0, total_v)
            fire(r0 + 2, 0)
            drain(1)
            total_v = compute(r0 + 1, 1, total_v)
            return total_v

        total_v = lax.fori_loop(0, (rpw - 1) // 2, pair_body,
                                jnp.zeros((_L,), jnp.float32))
        drain(0)
        total_v = compute(rpw - 1, 0, total_v)

        stage[...] = total_v
        pltpu.sync_copy(stage, out.at[wid])

    return tl_kernel


_NBLK = 1024


def _row_norms(h2, h3):
    n, d = h2.shape
    npad = -(-n // _NBLK) * _NBLK
    h2p = jnp.pad(h2, ((0, npad - n), (0, 0)))
    h3p = jnp.pad(h3, ((0, npad - n), (0, 0)))

    def body(a_ref, b_ref, o_ref):
        a = a_ref[...]
        b = b_ref[...]
        o_ref[0, :] = jnp.sum(a * a, axis=1)
        o_ref[1, :] = jnp.sum(b * b, axis=1)

    return pl.pallas_call(
        body,
        grid=(npad // _NBLK,),
        in_specs=[pl.BlockSpec((_NBLK, d), lambda i: (i, 0)),
                  pl.BlockSpec((_NBLK, d), lambda i: (i, 0))],
        out_specs=pl.BlockSpec((2, _NBLK), lambda i: (0, i)),
        out_shape=jax.ShapeDtypeStruct((2, npad), jnp.float32),
    )(h2p, h3p)


def kernel(h_c1, h_c2, h_c3, triplets):
    n_rows, d = h_c1.shape
    t = triplets.shape[0]
    tr = triplets.astype(jnp.int32)
    g_base, g_rem, rpw, iw = _plan(t)
    def _packbf(h):
        hb = h.astype(jnp.bfloat16).reshape(n_rows, d // 2, 2)
        return lax.bitcast_convert_type(hb, jnp.int32)
    b1, b2, b3 = _packbf(h_c1), _packbf(h_c2), _packbf(h_c3)
    padded = (g_base * (_NW - 1) + g_rem) * _L + iw
    pad = padded - t
    ia = jnp.pad(tr[:, 0], (0, pad))
    ip = jnp.pad(tr[:, 1], (0, pad))
    inn = jnp.pad(tr[:, 2], (0, pad))
    norms = _row_norms(h_c2, h_c3)
    partials = _make_kernel(n_rows, d, t, norms.shape[1])(
        b1, b2, b3, ia, ip, inn, norms)
    return jnp.sum(partials) / t + 1e-16


# speedup vs baseline: 2.7920x; 2.7920x over previous
import functools

import jax
import jax.numpy as jnp
from jax import lax
from jax.experimental import pallas as pl
from jax.experimental.pallas import tpu as pltpu
from jax.experimental.pallas import tpu_sc as plsc

_NC = 2
_NS = 16
_NW = _NC * _NS
_L = 16
_SUB = 4
_CH = _SUB * _L
_MARGIN = 1.0


def _plan(t):
    assert t % _L == 0
    ng_total = t // _L
    g_base = ng_total // _NW
    g_rem = ng_total % _NW
    maxg = g_base + (1 if g_rem else 0)
    rpw = -(-maxg // _SUB)
    if rpw % 2 == 0:
        rpw += 1
    iw = rpw * _CH
    for w in range(_NW):
        start_raw = (g_base * w + min(w, g_rem)) * _L
        ng = g_base + (1 if w < g_rem else 0)
        toff = max(0, start_raw - min(start_raw, t - iw)) // _L
        assert toff + ng <= rpw * _SUB, (w, toff, ng)
    return g_base, g_rem, rpw, iw


def _make_kernel(n_rows, d, t):
    assert d % 32 == 0
    dp = d // 2
    g_base, g_rem, rpw, iw = _plan(t)
    mesh = plsc.VectorSubcoreMesh(core_axis_name="c", subcore_axis_name="s")

    @functools.partial(
        pl.kernel,
        mesh=mesh,
        compiler_params=pltpu.CompilerParams(use_tc_tiling_on_sc=False,
                                             needs_layout_passes=False),
        out_type=jax.ShapeDtypeStruct((_NW, _L), jnp.float32),
        scratch_types=[
            pltpu.VMEM((iw,), jnp.int32),
            pltpu.VMEM((iw,), jnp.int32),
            pltpu.VMEM((iw,), jnp.int32),
            pltpu.VMEM((2, _CH, dp), jnp.int32),
            pltpu.VMEM((2, _CH, dp), jnp.int32),
            pltpu.VMEM((2, _CH, dp), jnp.int32),
            pltpu.VMEM((n_rows,), jnp.float32),
            pltpu.VMEM((n_rows,), jnp.float32),
            pltpu.VMEM((_L,), jnp.float32),
            pltpu.SemaphoreType.DMA,
            pltpu.SemaphoreType.DMA,
        ],
    )
    def tl_kernel(h1, h2, h3, ia, ip, inn, norms, out,
                  ixa, ixp, ixn, ra, rp, rn, n2v, n3v, stage, sem0, sem1):
        wid = lax.axis_index("s") * _NC + lax.axis_index("c")
        ngroups = g_base + jnp.where(wid < g_rem, 1, 0)
        start_raw = (g_base * wid + jnp.minimum(wid, g_rem)) * _L
        start = jnp.minimum(start_raw, t - iw)
        toff = (start_raw - start) // _L

        pltpu.sync_copy(ia.at[pl.ds(start, iw)], ixa)
        pltpu.sync_copy(ip.at[pl.ds(start, iw)], ixp)
        pltpu.sync_copy(inn.at[pl.ds(start, iw)], ixn)
        pltpu.sync_copy(norms.at[0], n2v)
        pltpu.sync_copy(norms.at[1], n3v)

        lanes = lax.iota(jnp.int32, _L)
        sems = (sem0, sem1)

        def fire(r, b):
            rb = r * _CH
            pltpu.async_copy(h1.at[ixa.at[pl.ds(rb, _CH)]], ra.at[b], sems[b])
            pltpu.async_copy(h2.at[ixp.at[pl.ds(rb, _CH)]], rp.at[b], sems[b])
            pltpu.async_copy(h3.at[ixn.at[pl.ds(rb, _CH)]], rn.at[b], sems[b])

        def drain(b):
            for buf in (ra, rp, rn):
                pltpu.make_async_copy(
                    h1.at[ixa.at[pl.ds(0, _CH)]], buf.at[b], sems[b]).wait()


        def compute(r, b, total_v):
            for sub in range(_SUB):
                rows = lanes + (sub * _L)
                ipv = ixp[pl.ds(r * _CH + sub * _L, _L)]
                inv = ixn[pl.ds(r * _CH + sub * _L, _L)]
                g2 = plsc.load_gather(n2v, [ipv])
                g3 = plsc.load_gather(n3v, [inv])
                accs = [jnp.zeros((_L,), jnp.float32) for _ in range(4)]

                def d_body(db, accs, _rows=rows):
                    accs = list(accs)
                    basev = jnp.full((_L,), db * _L, jnp.int32)
                    def _halves(v):
                        hi = plsc.bitcast(v, jnp.float32)
                        lo = plsc.bitcast(v << 16, jnp.float32)
                        return hi, lo

                    for s in range(_L):
                        dv = basev + (lanes ^ s)
                        a0, a1 = _halves(plsc.load_gather(ra.at[b], [_rows, dv]))
                        p0, p1 = _halves(plsc.load_gather(rp.at[b], [_rows, dv]))
                        n0, n1 = _halves(plsc.load_gather(rn.at[b], [_rows, dv]))
                        k = (s % 2) * 2
                        accs[k] = accs[k] + a0 * (p0 - n0)
                        accs[k + 1] = accs[k + 1] + a1 * (p1 - n1)
                    return tuple(accs)

                accs = lax.fori_loop(0, dp // _L, d_body, tuple(accs))
                dot = (accs[0] + accs[1]) + (accs[2] + accs[3])
                lossv = jnp.maximum(g2 - g3 - dot - dot + _MARGIN, 0.0)
                g_loc = r * _SUB + sub
                gate = ((g_loc >= toff) & (g_loc < toff + ngroups)
                        ).astype(jnp.float32)
                total_v = total_v + lossv * gate
            return total_v

        fire(0, 0)

        def pair_body(k, total_v):
            r0 = 2 * k
            fire(r0 + 1, 1)
            drain(0)
            total_v = compute(r0, 0, total_v)
            fire(r0 + 2, 0)
            drain(1)
            total_v = compute(r0 + 1, 1, total_v)
            return total_v

        total_v = lax.fori_loop(0, (rpw - 1) // 2, pair_body,
                                jnp.zeros((_L,), jnp.float32))
        drain(0)
        total_v = compute(rpw - 1, 0, total_v)

        stage[...] = total_v
        pltpu.sync_copy(stage, out.at[wid])

    return tl_kernel


def _prep(h1, h2, h3):
    n, d = h1.shape
    dh = d // 2

    def body(x1, x2, x3, p1, p2, p3, nrm):
        for xr, pr in ((x1, p1), (x2, p2), (x3, p3)):
            x = xr[...]
            lo = x[:, :dh].astype(jnp.bfloat16).astype(jnp.float32)
            hi = x[:, dh:].astype(jnp.bfloat16).astype(jnp.float32)
            ulo = lax.bitcast_convert_type(lo, jnp.uint32) >> 16
            uhi = lax.bitcast_convert_type(hi, jnp.uint32)
            pr[...] = lax.bitcast_convert_type(ulo | uhi, jnp.int32)
        x2v = x2[...]
        x3v = x3[...]
        nrm[0, :] = jnp.sum(x2v * x2v, axis=1)
        nrm[1, :] = jnp.sum(x3v * x3v, axis=1)

    return pl.pallas_call(
        body,
        compiler_params=pltpu.CompilerParams(vmem_limit_bytes=100_000_000),
        out_shape=[jax.ShapeDtypeStruct((n, dh), jnp.int32)] * 3
        + [jax.ShapeDtypeStruct((2, n), jnp.float32)],
    )(h1, h2, h3)


def kernel(h_c1, h_c2, h_c3, triplets):
    n_rows, d = h_c1.shape
    t = triplets.shape[0]
    tr = triplets.astype(jnp.int32)
    _plan(t)
    b1, b2, b3, norms = _prep(h_c1, h_c2, h_c3)
    partials = _make_kernel(n_rows, d, t)(
        b1, b2, b3, tr[:, 0], tr[:, 1], tr[:, 2], norms)
    return jnp.sum(partials) / t + 1e-16
